# Initial kernel scaffold; baseline (speedup 1.0000x reference)
import jax
import jax.numpy as jnp
from jax.experimental import pallas as pl


def kernel(tokens, values, emb_a, emb_b, proj_w):
    raise NotImplementedError("write your pallas kernel here")


# speedup vs baseline: 3.1566x; 3.1566x over previous
import functools

import jax
import jax.numpy as jnp
from jax.experimental import pallas as pl
from jax.experimental.pallas import tpu as pltpu

_NFD = 4
_MASK_ID = 1
_SOS_ID = 2
_EOS_ID = 3
_EMB = 64
_HALF = _EMB // 2
_PROJ = 512
_BB = 8


def _shift(x, k, fill):
    pad = jnp.full(x.shape[:-1] + (k,), fill, x.dtype)
    return jnp.concatenate([pad, x[..., : x.shape[-1] - k]], axis=-1)


def _process_dim(t, v):
    s_len = t.shape[-1]
    sp0 = t <= _NFD
    tokc = jnp.where(sp0, t, _NFD)
    v = jnp.where(sp0, jnp.float32(0.0), v)
    sp2 = sp0 & (t != _SOS_ID) & (t != _EOS_ID)

    u = sp2.astype(jnp.int32)
    ck = v
    a = jnp.where(sp2, jnp.float32(1e-6), jnp.float32(1.0))
    b = jnp.where(v < 0, jnp.float32(-1.0), jnp.float32(1.0)) * jnp.maximum(
        jnp.abs(v), jnp.float32(1e-6)
    )
    k = 1
    while k < s_len:
        u = u | _shift(u, k, 0)
        ck = ck + _shift(ck, k, jnp.float32(0.0))
        a_sh = _shift(a, k, jnp.float32(1.0))
        b_sh = _shift(b, k, jnp.float32(0.0))
        b = a * b_sh + b
        a = a * a_sh
        k *= 2

    unk = u > 0
    pos_known = jnp.where(unk, jnp.float32(0.0), ck)
    tok_known = jnp.where(unk & (tokc == _NFD), _MASK_ID, tokc)
    pos_int = jnp.round(b, 4)
    return tokc, tok_known, pos_known, pos_int


def _fwd_kernel(t0_ref, t1_ref, v0_ref, v1_ref, ea_ref, eb_ref, w_ref, o_ref):
    bb, s_len = t0_ref.shape
    tc0, tk0, pk0, pi0 = _process_dim(t0_ref[...], v0_ref[...])
    tc1, tk1, pk1, pi1 = _process_dim(t1_ref[...], v1_ref[...])

    freqs = jnp.exp(
        -jnp.log(jnp.float32(10000.0))
        * jax.lax.broadcasted_iota(jnp.int32, (1, 1, _HALF), 2).astype(jnp.float32)
        / _HALF
    )

    def sincos(pos):
        ang = pos[:, :, None] * freqs
        return jnp.concatenate([jnp.sin(ang), jnp.cos(ang)], axis=-1)

    feat = jnp.concatenate(
        [sincos(pk0), sincos(pk1), sincos(pi0), sincos(pi1)], axis=-1
    ).reshape(bb * s_len, 4 * _EMB)

    iota5 = jax.lax.broadcasted_iota(jnp.int32, (1, 1, _NFD + 1), 2)

    def onehot(tok):
        return (tok[:, :, None] == iota5).astype(jnp.float32)

    oh = jnp.concatenate(
        [onehot(tk0), onehot(tk1), onehot(tc0), onehot(tc1)], axis=-1
    ).reshape(bb * s_len, 4 * (_NFD + 1))

    w = w_ref[...]
    ea = ea_ref[...]
    eb = eb_ref[...]
    tall = jnp.concatenate(
        [
            jnp.dot(ea, w[0 * _EMB : 1 * _EMB], preferred_element_type=jnp.float32),
            jnp.dot(eb, w[1 * _EMB : 2 * _EMB], preferred_element_type=jnp.float32),
            jnp.dot(ea, w[2 * _EMB : 3 * _EMB], preferred_element_type=jnp.float32),
            jnp.dot(eb, w[3 * _EMB : 4 * _EMB], preferred_element_type=jnp.float32),
        ],
        axis=0,
    )

    y = jnp.dot(feat, w, preferred_element_type=jnp.float32) + jnp.dot(
        oh, tall, preferred_element_type=jnp.float32
    )
    o_ref[...] = y.reshape(bb, s_len, _PROJ)


@functools.partial(jax.jit, static_argnames=())
def kernel(tokens, values, emb_a, emb_b, proj_w):
    b, s, _ = tokens.shape
    tok0 = tokens[..., 0]
    tok1 = tokens[..., 1]
    val0 = values[..., 0]
    val1 = values[..., 1]
    wp = proj_w.T

    seq_spec = pl.BlockSpec((_BB, s), lambda i: (i, 0))
    full = lambda shape: pl.BlockSpec(shape, lambda i: (0,) * len(shape))

    return pl.pallas_call(
        _fwd_kernel,
        grid=(b // _BB,),
        in_specs=[
            seq_spec,
            seq_spec,
            seq_spec,
            seq_spec,
            full(emb_a.shape),
            full(emb_b.shape),
            full(wp.shape),
        ],
        out_specs=pl.BlockSpec((_BB, s, _PROJ), lambda i: (i, 0, 0)),
        out_shape=jax.ShapeDtypeStruct((b, s, _PROJ), jnp.float32),
        compiler_params=pltpu.CompilerParams(
            dimension_semantics=("parallel",),
        ),
    )(tok0, tok1, val0, val1, emb_a, emb_b, wp)


# speedup vs baseline: 9.0898x; 2.8796x over previous
import functools

import jax
import jax.numpy as jnp
from jax.experimental import pallas as pl
from jax.experimental.pallas import tpu as pltpu

_NFD = 4
_MASK_ID = 1
_SOS_ID = 2
_EOS_ID = 3
_EMB = 64
_HALF = _EMB // 2
_PROJ = 512
_BB = 8


def _shift(x, k, fill):
    pad = jnp.full(x.shape[:-1] + (k,), fill, x.dtype)
    return jnp.concatenate([pad, x[..., : x.shape[-1] - k]], axis=-1)


def _process_dim(t, v):
    s_len = t.shape[-1]
    sp0 = t <= _NFD
    tokc = jnp.where(sp0, t, _NFD)
    v = jnp.where(sp0, jnp.float32(0.0), v)
    sp2 = sp0 & (t != _SOS_ID) & (t != _EOS_ID)

    u = sp2.astype(jnp.int32)
    ck = v
    a = jnp.where(sp2, jnp.float32(1e-6), jnp.float32(1.0))
    b = jnp.where(v < 0, jnp.float32(-1.0), jnp.float32(1.0)) * jnp.maximum(
        jnp.abs(v), jnp.float32(1e-6)
    )
    k = 1
    while k < s_len:
        u = u | _shift(u, k, 0)
        ck = ck + _shift(ck, k, jnp.float32(0.0))
        a_sh = _shift(a, k, jnp.float32(1.0))
        b_sh = _shift(b, k, jnp.float32(0.0))
        b = a * b_sh + b
        a = a * a_sh
        k *= 2

    unk = u > 0
    pos_known = jnp.where(unk, jnp.float32(0.0), ck)
    tok_known = jnp.where(unk & (tokc == _NFD), _MASK_ID, tokc)
    pos_int = jnp.round(b, 4)
    return tokc, tok_known, pos_known, pos_int


def _fwd_kernel(t0_ref, t1_ref, v0_ref, v1_ref, ea_ref, eb_ref, w_ref, o_ref):
    bb, s_len = t0_ref.shape
    tc0, tk0, pk0, pi0 = _process_dim(t0_ref[...], v0_ref[...])
    tc1, tk1, pk1, pi1 = _process_dim(t1_ref[...], v1_ref[...])

    freqs = jnp.exp(
        -jnp.log(jnp.float32(10000.0))
        * jax.lax.broadcasted_iota(jnp.int32, (_HALF, 1), 0).astype(jnp.float32)
        / _HALF
    )
    iota5 = jax.lax.broadcasted_iota(jnp.int32, (_NFD + 1, 1), 0)

    w = w_ref[...]
    ea = ea_ref[...]
    eb = eb_ref[...]
    tall = jnp.concatenate(
        [
            jnp.dot(ea, w[0 * _EMB : 1 * _EMB], preferred_element_type=jnp.float32),
            jnp.dot(eb, w[1 * _EMB : 2 * _EMB], preferred_element_type=jnp.float32),
            jnp.dot(ea, w[2 * _EMB : 3 * _EMB], preferred_element_type=jnp.float32),
            jnp.dot(eb, w[3 * _EMB : 4 * _EMB], preferred_element_type=jnp.float32),
        ],
        axis=0,
    )

    dnums_t = (((0,), (0,)), ((), ()))
    for b in range(bb):
        parts = []
        for pos in (pk0[b : b + 1], pk1[b : b + 1], pi0[b : b + 1], pi1[b : b + 1]):
            ang = freqs * pos
            parts.append(jnp.sin(ang))
            parts.append(jnp.cos(ang))
        feat_t = jnp.concatenate(parts, axis=0)
        oh_t = jnp.concatenate(
            [
                (tk0[b : b + 1] == iota5).astype(jnp.float32),
                (tk1[b : b + 1] == iota5).astype(jnp.float32),
                (tc0[b : b + 1] == iota5).astype(jnp.float32),
                (tc1[b : b + 1] == iota5).astype(jnp.float32),
            ],
            axis=0,
        )
        y = jax.lax.dot_general(
            feat_t, w, dnums_t, preferred_element_type=jnp.float32
        ) + jax.lax.dot_general(oh_t, tall, dnums_t, preferred_element_type=jnp.float32)
        o_ref[b] = y


@functools.partial(jax.jit, static_argnames=())
def kernel(tokens, values, emb_a, emb_b, proj_w):
    b, s, _ = tokens.shape
    tok0 = tokens[..., 0]
    tok1 = tokens[..., 1]
    val0 = values[..., 0]
    val1 = values[..., 1]
    wp = proj_w.T

    seq_spec = pl.BlockSpec((_BB, s), lambda i: (i, 0))
    full = lambda shape: pl.BlockSpec(shape, lambda i: (0,) * len(shape))

    return pl.pallas_call(
        _fwd_kernel,
        grid=(b // _BB,),
        in_specs=[
            seq_spec,
            seq_spec,
            seq_spec,
            seq_spec,
            full(emb_a.shape),
            full(emb_b.shape),
            full(wp.shape),
        ],
        out_specs=pl.BlockSpec((_BB, s, _PROJ), lambda i: (i, 0, 0)),
        out_shape=jax.ShapeDtypeStruct((b, s, _PROJ), jnp.float32),
        compiler_params=pltpu.CompilerParams(
            dimension_semantics=("parallel",),
        ),
    )(tok0, tok1, val0, val1, emb_a, emb_b, wp)


# speedup vs baseline: 20.2055x; 2.2229x over previous
import functools

import jax
import jax.numpy as jnp
from jax.experimental import pallas as pl
from jax.experimental.pallas import tpu as pltpu

_NFD = 4
_MASK_ID = 1
_SOS_ID = 2
_EOS_ID = 3
_EMB = 64
_HALF = _EMB // 2
_PROJ = 512
_BB = 8


def _shift(x, k, fill):
    pad = jnp.full(x.shape[:-1] + (k,), fill, x.dtype)
    return jnp.concatenate([pad, x[..., : x.shape[-1] - k]], axis=-1)


def _process_dim(t, v):
    s_len = t.shape[-1]
    sp0 = t <= _NFD
    tokc = jnp.where(sp0, t, _NFD)
    v = jnp.where(sp0, jnp.float32(0.0), v)
    sp2 = sp0 & (t != _SOS_ID) & (t != _EOS_ID)

    u = sp2.astype(jnp.int32)
    ck = v
    a = jnp.where(sp2, jnp.float32(1e-6), jnp.float32(1.0))
    b = jnp.where(v < 0, jnp.float32(-1.0), jnp.float32(1.0)) * jnp.maximum(
        jnp.abs(v), jnp.float32(1e-6)
    )
    k = 1
    while k < s_len:
        u = u | _shift(u, k, 0)
        ck = ck + _shift(ck, k, jnp.float32(0.0))
        a_sh = _shift(a, k, jnp.float32(1.0))
        b_sh = _shift(b, k, jnp.float32(0.0))
        b = a * b_sh + b
        a = a * a_sh
        k *= 2

    unk = u > 0
    pos_known = jnp.where(unk, jnp.float32(0.0), ck)
    tok_known = jnp.where(unk & (tokc == _NFD), _MASK_ID, tokc)
    pos_int = jnp.round(b, 4)
    return tokc, tok_known, pos_known, pos_int


_SINP = (0.9999999957147785, -0.1666665796818604, 0.008333050575534767,
         -0.00019809043195644972, 2.60515895350761e-06)
_COSP = (0.9999999532360521, -0.4999990504408244, 0.04166357847528418,
         -0.0013853663487688448, 2.3153094364380012e-05)
_PI = 3.14159265358979
_INV_PI = 0.3183098861837907


def _fast_sincos(ang):
    f32 = jnp.float32
    n = jnp.round(ang * f32(_INV_PI))
    r = ang - n * f32(_PI)
    odd = (n.astype(jnp.int32) & 1) != 0
    sign = jnp.where(odd, f32(-1.0), f32(1.0))
    t = r * r
    s0, s1, s2, s3, s4 = (f32(c) for c in _SINP)
    c0, c1, c2, c3, c4 = (f32(c) for c in _COSP)
    sin_r = r * (s0 + t * (s1 + t * (s2 + t * (s3 + t * s4))))
    cos_r = c0 + t * (c1 + t * (c2 + t * (c3 + t * c4)))
    return sign * sin_r, sign * cos_r


def _fwd_kernel(t0_ref, t1_ref, v0_ref, v1_ref, ea_ref, eb_ref, w_ref, o_ref):
    bb, s_len = t0_ref.shape
    tc0, tk0, pk0, pi0 = _process_dim(t0_ref[...], v0_ref[...])
    tc1, tk1, pk1, pi1 = _process_dim(t1_ref[...], v1_ref[...])

    freqs = jnp.exp(
        -jnp.log(jnp.float32(10000.0))
        * jax.lax.broadcasted_iota(jnp.int32, (_HALF, 1), 0).astype(jnp.float32)
        / _HALF
    )
    iota5 = jax.lax.broadcasted_iota(jnp.int32, (_NFD + 1, 1), 0)

    w = w_ref[...]
    ea = ea_ref[...]
    eb = eb_ref[...]
    tall = jnp.concatenate(
        [
            jnp.dot(ea, w[0 * _EMB : 1 * _EMB], preferred_element_type=jnp.float32),
            jnp.dot(eb, w[1 * _EMB : 2 * _EMB], preferred_element_type=jnp.float32),
            jnp.dot(ea, w[2 * _EMB : 3 * _EMB], preferred_element_type=jnp.float32),
            jnp.dot(eb, w[3 * _EMB : 4 * _EMB], preferred_element_type=jnp.float32),
        ],
        axis=0,
    )

    dnums_t = (((0,), (0,)), ((), ()))
    for b in range(bb):
        parts = []
        for pos in (pk0[b : b + 1], pk1[b : b + 1], pi0[b : b + 1], pi1[b : b + 1]):
            ang = freqs * pos
            sin_a, cos_a = _fast_sincos(ang)
            parts.append(sin_a)
            parts.append(cos_a)
        feat_t = jnp.concatenate(parts, axis=0)
        oh_t = jnp.concatenate(
            [
                (tk0[b : b + 1] == iota5).astype(jnp.float32),
                (tk1[b : b + 1] == iota5).astype(jnp.float32),
                (tc0[b : b + 1] == iota5).astype(jnp.float32),
                (tc1[b : b + 1] == iota5).astype(jnp.float32),
            ],
            axis=0,
        )
        y = jax.lax.dot_general(
            feat_t, w, dnums_t, preferred_element_type=jnp.float32
        ) + jax.lax.dot_general(oh_t, tall, dnums_t, preferred_element_type=jnp.float32)
        o_ref[b] = y


@functools.partial(jax.jit, static_argnames=())
def kernel(tokens, values, emb_a, emb_b, proj_w):
    b, s, _ = tokens.shape
    tok0 = tokens[..., 0]
    tok1 = tokens[..., 1]
    val0 = values[..., 0]
    val1 = values[..., 1]
    wp = proj_w.T

    seq_spec = pl.BlockSpec((_BB, s), lambda i: (i, 0))
    full = lambda shape: pl.BlockSpec(shape, lambda i: (0,) * len(shape))

    return pl.pallas_call(
        _fwd_kernel,
        grid=(b // _BB,),
        in_specs=[
            seq_spec,
            seq_spec,
            seq_spec,
            seq_spec,
            full(emb_a.shape),
            full(emb_b.shape),
            full(wp.shape),
        ],
        out_specs=pl.BlockSpec((_BB, s, _PROJ), lambda i: (i, 0, 0)),
        out_shape=jax.ShapeDtypeStruct((b, s, _PROJ), jnp.float32),
        compiler_params=pltpu.CompilerParams(
            dimension_semantics=("parallel",),
        ),
    )(tok0, tok1, val0, val1, emb_a, emb_b, wp)
